# Initial kernel scaffold; baseline (speedup 1.0000x reference)
import jax
import jax.numpy as jnp
from jax.experimental import pallas as pl


def kernel(x, edge_index, edge_attr, batch, params):
    raise NotImplementedError("write your pallas kernel here")


# speedup vs baseline: 3.4469x; 3.4469x over previous
import functools

import jax
import jax.numpy as jnp
from jax import lax
from jax.experimental import pallas as pl
from jax.experimental.pallas import tpu as pltpu
from jax.experimental.pallas import tpu_sc as plsc

N = 10000
E = 320000
D = 128
G = 64
L = 5
C = 2

RB = 1000
NBLK = N // RB

NC = 2
NS = 16
NW = NC * NS
EPW = E // NW
EB = 80
NCHUNK = EPW // EB
RPT = 624
REM = N - RPT * NS
ZB = 104


def _sc_message_body(hin, src, dst, ea, out, acc, idx_v, didx_v, rows_v, ea_v,
                     zbuf, sem):
    c = lax.axis_index("c")
    s = lax.axis_index("s")
    wid = c * NS + s

    zero = jnp.zeros((16,), jnp.float32)

    def zfill(i, carry):
        r = i // 8
        k = (i % 8) * 16
        zbuf[r, pl.ds(k, 16)] = zero
        return carry

    lax.fori_loop(0, ZB * 8, zfill, 0)
    for j in range(RPT // ZB):
        pltpu.sync_copy(zbuf, acc.at[pl.ds(s * RPT + j * ZB, ZB)])

    @pl.when(s == 0)
    def _():
        pltpu.sync_copy(zbuf.at[pl.ds(0, REM)], acc.at[pl.ds(RPT * NS, REM)])

    plsc.subcore_barrier()

    def chunk(ci, carry):
        base = wid * EPW + ci * EB
        pltpu.sync_copy(src.at[pl.ds(base, EB)], idx_v)
        pltpu.sync_copy(dst.at[pl.ds(base, EB)], didx_v)
        gcp = pltpu.async_copy(hin.at[idx_v], rows_v, sem)
        pltpu.sync_copy(ea.at[pl.ds(base, EB)], ea_v)
        gcp.wait()

        def rowop(r, rcarry):
            for kk in range(8):
                sl = pl.ds(kk * 16, 16)
                rows_v[r, sl] = jnp.maximum(rows_v[r, sl] + ea_v[r, sl], 0.0)
            return rcarry

        lax.fori_loop(0, EB, rowop, 0)
        pltpu.sync_copy(rows_v, acc.at[didx_v], add=True)
        return carry

    lax.fori_loop(0, NCHUNK, chunk, 0)
    plsc.subcore_barrier()

    pltpu.sync_copy(acc.at[pl.ds(s * RPT, RPT)],
                    out.at[c, pl.ds(s * RPT, RPT)])

    @pl.when(s == 0)
    def _():
        pltpu.sync_copy(acc.at[pl.ds(RPT * NS, REM)],
                        out.at[c, pl.ds(RPT * NS, REM)])


@jax.jit
def _sc_message(hin, src, dst, ea):
    mesh = plsc.VectorSubcoreMesh(core_axis_name="c", subcore_axis_name="s",
                                  num_cores=NC, num_subcores=NS)
    return pl.kernel(
        _sc_message_body,
        out_type=jax.ShapeDtypeStruct((NC, N, D), jnp.float32),
        mesh=mesh,
        scratch_types=[
            pltpu.VMEM_SHARED((N, D), jnp.float32),
            pltpu.VMEM((EB,), jnp.int32),
            pltpu.VMEM((EB,), jnp.int32),
            pltpu.VMEM((EB, D), jnp.float32),
            pltpu.VMEM((EB, D), jnp.float32),
            pltpu.VMEM((ZB, D), jnp.float32),
            pltpu.SemaphoreType.DMA,
        ],
    )(hin, src, dst, ea)


def _tc_pre_body(h_ref, vn_ref, b_ref, hin_ref, seg_ref):
    i = pl.program_id(0)
    bk = b_ref[0]
    onehot = (bk.T == lax.broadcasted_iota(jnp.int32, (RB, G), 1)
              ).astype(jnp.float32)
    hin = h_ref[...] + jnp.dot(onehot, vn_ref[...],
                               preferred_element_type=jnp.float32)
    hin_ref[...] = hin
    contrib = jnp.dot(onehot.T, hin, preferred_element_type=jnp.float32)

    @pl.when(i == 0)
    def _():
        seg_ref[...] = contrib

    @pl.when(i != 0)
    def _():
        seg_ref[...] += contrib


@jax.jit
def _tc_pre(h, vn, batch3):
    return pl.pallas_call(
        _tc_pre_body,
        grid=(NBLK,),
        in_specs=[
            pl.BlockSpec((RB, D), lambda i: (i, 0)),
            pl.BlockSpec((G, D), lambda i: (0, 0)),
            pl.BlockSpec((1, 1, RB), lambda i: (i, 0, 0)),
        ],
        out_specs=[
            pl.BlockSpec((RB, D), lambda i: (i, 0)),
            pl.BlockSpec((G, D), lambda i: (0, 0)),
        ],
        out_shape=[
            jax.ShapeDtypeStruct((N, D), jnp.float32),
            jax.ShapeDtypeStruct((G, D), jnp.float32),
        ],
    )(h, vn, batch3)


def _tc_mlp_body(hin_ref, agg_ref, eps_ref, w1_ref, a1_ref, c1_ref, w2_ref,
                 a2_ref, c2_ref, out_ref, *, final_relu):
    z = (1.0 + eps_ref[0, 0]) * hin_ref[...] + agg_ref[0] + agg_ref[1]
    z1 = jnp.dot(z, w1_ref[...].T, preferred_element_type=jnp.float32)
    z1 = z1 * a1_ref[...] + c1_ref[...]
    z1 = jnp.maximum(z1, 0.0)
    z2 = jnp.dot(z1, w2_ref[...].T, preferred_element_type=jnp.float32)
    z2 = z2 * a2_ref[...] + c2_ref[...]
    if final_relu:
        z2 = jnp.maximum(z2, 0.0)
    out_ref[...] = z2


@functools.partial(jax.jit, static_argnames=("final_relu",))
def _tc_mlp(hin, agg2, eps, w1, a1, c1, w2, a2, c2, *, final_relu):
    body = functools.partial(_tc_mlp_body, final_relu=final_relu)
    return pl.pallas_call(
        body,
        grid=(NBLK,),
        in_specs=[
            pl.BlockSpec((RB, D), lambda i: (i, 0)),
            pl.BlockSpec((NC, RB, D), lambda i: (0, i, 0)),
            pl.BlockSpec((1, 1), lambda i: (0, 0)),
            pl.BlockSpec((2 * D, D), lambda i: (0, 0)),
            pl.BlockSpec((1, 2 * D), lambda i: (0, 0)),
            pl.BlockSpec((1, 2 * D), lambda i: (0, 0)),
            pl.BlockSpec((D, 2 * D), lambda i: (0, 0)),
            pl.BlockSpec((1, D), lambda i: (0, 0)),
            pl.BlockSpec((1, D), lambda i: (0, 0)),
        ],
        out_specs=pl.BlockSpec((RB, D), lambda i: (i, 0)),
        out_shape=jax.ShapeDtypeStruct((N, D), jnp.float32),
    )(hin, agg2, eps, w1, a1, c1, w2, a2, c2)


def _tc_vn_body(seg_ref, vn_ref, w1_ref, a1_ref, c1_ref, w2_ref, a2_ref,
                c2_ref, out_ref):
    vt = seg_ref[...] + vn_ref[...]
    v = jnp.dot(vt, w1_ref[...].T, preferred_element_type=jnp.float32)
    v = v * a1_ref[...] + c1_ref[...]
    v = jnp.maximum(v, 0.0)
    v = jnp.dot(v, w2_ref[...].T, preferred_element_type=jnp.float32)
    v = v * a2_ref[...] + c2_ref[...]
    out_ref[...] = jnp.maximum(v, 0.0)


@jax.jit
def _tc_vn(seg, vn, w1, a1, c1, w2, a2, c2):
    return pl.pallas_call(
        _tc_vn_body,
        out_shape=jax.ShapeDtypeStruct((G, D), jnp.float32),
    )(seg, vn, w1, a1, c1, w2, a2, c2)


def _tc_pool_body(h_ref, b_ref, wh_ref, bh_ref, out_ref, sum_ref, cnt_ref):
    i = pl.program_id(0)
    bk = b_ref[0]
    onehot = (bk.T == lax.broadcasted_iota(jnp.int32, (RB, G), 1)
              ).astype(jnp.float32)

    @pl.when(i == 0)
    def _():
        sum_ref[...] = jnp.zeros((G, D), jnp.float32)
        cnt_ref[...] = jnp.zeros((G, D), jnp.float32)

    sum_ref[...] += jnp.dot(onehot.T, h_ref[...],
                            preferred_element_type=jnp.float32)
    cnt_ref[...] += jnp.dot(onehot.T, jnp.ones((RB, D), jnp.float32),
                            preferred_element_type=jnp.float32)

    @pl.when(i == NBLK - 1)
    def _():
        hg = sum_ref[...] / jnp.maximum(cnt_ref[...], 1.0)
        out_ref[...] = jnp.dot(hg, wh_ref[...].T,
                               preferred_element_type=jnp.float32) + bh_ref[...]


@jax.jit
def _tc_pool(h, batch3, wh_pad, bh_pad):
    return pl.pallas_call(
        _tc_pool_body,
        grid=(NBLK,),
        in_specs=[
            pl.BlockSpec((RB, D), lambda i: (i, 0)),
            pl.BlockSpec((1, 1, RB), lambda i: (i, 0, 0)),
            pl.BlockSpec((D, D), lambda i: (0, 0)),
            pl.BlockSpec((1, D), lambda i: (0, 0)),
        ],
        out_specs=pl.BlockSpec((G, D), lambda i: (0, 0)),
        out_shape=jax.ShapeDtypeStruct((G, D), jnp.float32),
        scratch_shapes=[
            pltpu.VMEM((G, D), jnp.float32),
            pltpu.VMEM((G, D), jnp.float32),
        ],
    )(h, batch3, wh_pad, bh_pad)


def kernel(x, edge_index, edge_attr, batch, params):
    src = edge_index[0]
    dst = edge_index[1]
    batch3 = batch.reshape(NBLK, 1, RB)

    p = params
    wh_pad = jnp.zeros((D, D), jnp.float32).at[:C, :].set(p['Wh'])
    bh_pad = jnp.zeros((1, D), jnp.float32).at[0, :C].set(p['bh'])

    h = x
    vn = jnp.zeros((G, D), jnp.float32)
    for l in range(L):
        hin, seg = _tc_pre(h, vn, batch3)
        agg2 = _sc_message(hin, src, dst, edge_attr)
        h = _tc_mlp(
            hin, agg2,
            p['eps'][l].reshape(1, 1),
            p['W1'][l], p['bn1_g'][l].reshape(1, 2 * D),
            (p['b1'][l] * p['bn1_g'][l] + p['bn1_b'][l]).reshape(1, 2 * D),
            p['W2'][l], p['bn_g'][l].reshape(1, D),
            (p['b2'][l] * p['bn_g'][l] + p['bn_b'][l]).reshape(1, D),
            final_relu=(l < L - 1),
        )
        if l < L - 1:
            vn = _tc_vn(
                seg, vn,
                p['vW1'][l], p['vbn1_g'][l].reshape(1, 2 * D),
                (p['vb1'][l] * p['vbn1_g'][l] + p['vbn1_b'][l]).reshape(1, 2 * D),
                p['vW2'][l], p['vbn2_g'][l].reshape(1, D),
                (p['vb2'][l] * p['vbn2_g'][l] + p['vbn2_b'][l]).reshape(1, D),
            )
    out = _tc_pool(h, batch3, wh_pad, bh_pad)
    return out[:, :C]
